# baseline (speedup 1.0000x reference)
import functools

import jax
import jax.numpy as jnp
from jax import lax
from jax.experimental import pallas as pl
from jax.experimental.pallas import tpu as pltpu
from jax.experimental.pallas import tpu_sc as plsc

_NC = 2
_NS = 16
_NW = _NC * _NS
_G = 8


_CB = 256


def _transpose_table(table_t, tail_packed):
    dm, v = table_t.shape
    nblk = v // _CB
    per = nblk // _NW
    nextra = nblk - per * _NW
    tail = v - nblk * _CB
    mesh = plsc.VectorSubcoreMesh(core_axis_name="c", subcore_axis_name="s")

    @functools.partial(
        pl.kernel,
        mesh=mesh,
        compiler_params=pltpu.CompilerParams(needs_layout_passes=False),
        out_type=jax.ShapeDtypeStruct((v * dm // 128, 128), table_t.dtype),
        scratch_types=[
            pltpu.VMEM((2, dm, _CB), jnp.float32),
            pltpu.VMEM((2, _CB * dm // 128, 128), jnp.float32),
        ]
        + [pltpu.SemaphoreType.DMA] * 4,
    )
    def _tp(tab_hbm, tailp_hbm, out_hbm, in_v, out_v, isem0, isem1, osem0, osem1):
        isems = (isem0, isem1)
        osems = (osem0, osem1)
        wid = lax.axis_index("c") * _NS + lax.axis_index("s")
        iota16 = jnp.arange(16, dtype=jnp.int32)

        def issue_in(blk, buf):
            pltpu.async_copy(
                tab_hbm.at[:, pl.ds(blk * _CB, _CB)], in_v.at[buf], isems[buf]
            )

        def wait_in(buf):
            pltpu.make_async_copy(
                tab_hbm.at[:, pl.ds(0, _CB)], in_v.at[buf], isems[buf]
            ).wait()

        def transpose_blk(buf, nrow):
            @pl.loop(0, nrow)
            def _(p):
                c0 = p * 4
                for m in range(8):
                    colv = jnp.full((16,), 0, jnp.int32) + (c0 + m // 2)
                    out_v.at[buf, p, pl.ds(m * 16, 16)][...] = plsc.load_gather(
                        in_v.at[buf], [iota16 + 16 * (m % 2), colv]
                    )

        def issue_out(blk, buf):
            pltpu.async_copy(
                out_v.at[buf],
                out_hbm.at[pl.ds(blk * (_CB * dm // 128), _CB * dm // 128)],
                osems[buf],
            )

        def wait_outd(buf):
            pltpu.make_async_copy(
                out_v.at[buf],
                out_hbm.at[pl.ds(0, _CB * dm // 128)],
                osems[buf],
            ).wait()

        base = wid * per
        issue_in(base, 0)
        issue_in(base + 1, 1)
        wait_in(0)
        transpose_blk(0, _CB // 4)
        issue_out(base, 0)
        issue_in(base + 2, 0)
        wait_in(1)
        transpose_blk(1, _CB // 4)
        issue_out(base + 1, 1)
        issue_in(base + 3, 1)

        @pl.loop(2, per - 2, step=2)
        def _(t):
            wait_in(0)
            wait_outd(0)
            transpose_blk(0, _CB // 4)
            issue_out(base + t, 0)
            issue_in(base + t + 2, 0)
            wait_in(1)
            wait_outd(1)
            transpose_blk(1, _CB // 4)
            issue_out(base + t + 1, 1)
            issue_in(base + t + 3, 1)

        wait_in(0)
        wait_outd(0)
        transpose_blk(0, _CB // 4)
        issue_out(base + per - 2, 0)
        wait_in(1)
        wait_outd(1)
        transpose_blk(1, _CB // 4)
        issue_out(base + per - 1, 1)
        wait_outd(0)
        wait_outd(1)

        @pl.when(wid < nextra)
        def _():
            blk = per * _NW + wid
            issue_in(blk, 0)
            wait_in(0)
            transpose_blk(0, _CB // 4)
            issue_out(blk, 0)
            wait_outd(0)

        @pl.when(wid == nextra)
        def _():
            nr = tail * dm // 128
            pltpu.sync_copy(tailp_hbm, out_v.at[0, pl.ds(0, nr)])
            pltpu.sync_copy(
                out_v.at[0, pl.ds(0, nr)],
                out_hbm.at[pl.ds(nblk * _CB * dm // 128, nr)],
            )

    return _tp(table_t, tail_packed)


def kernel(indices, table):
    b, s = indices.shape
    d = table.shape[1]
    nd = d // 8
    nl = b // 128
    lpw = nl // _NW
    nunits = s * lpw
    idx_t = indices.T
    ntail = table.shape[0] % _CB
    tail_packed = table[table.shape[0] - ntail:].reshape(ntail * d // 128, 128)
    packed = _transpose_table(table.T, tail_packed)
    table_rm = packed.reshape(-1).reshape(table.shape)
    mesh = plsc.VectorSubcoreMesh(core_axis_name="c", subcore_axis_name="s")

    @functools.partial(
        pl.kernel,
        mesh=mesh,
        compiler_params=pltpu.CompilerParams(
            use_tc_tiling_on_sc=False, needs_layout_passes=False
        ),
        out_type=jax.ShapeDtypeStruct((s, nd, nl, 8, 128), table.dtype),
        scratch_types=[
            pltpu.VMEM((s, lpw * 128), jnp.int32),
            pltpu.VMEM((_G, 128, d), jnp.float32),
            pltpu.VMEM((2, nd, 8, 128), jnp.float32),
        ]
        + [pltpu.SemaphoreType.DMA] * (_G + 2),
    )
    def _gather(table_hbm, idx_hbm, out_hbm, idx_v, rows_v, tiles_v, *sems):
        gsems = sems[:_G]
        osems = sems[_G:]
        wid = lax.axis_index("c") * _NS + lax.axis_index("s")
        lbase = wid * lpw
        pltpu.sync_copy(idx_hbm.at[:, pl.ds(lbase * 128, lpw * 128)], idx_v)

        def issue_gather(u, g):
            slot, j = u // lpw, u % lpw
            pltpu.async_copy(
                table_hbm.at[idx_v.at[slot].at[pl.ds(j * 128, 128)]],
                rows_v.at[g],
                gsems[g],
            )

        def drain_gather(g):
            pltpu.make_async_copy(
                table_hbm.at[idx_v.at[0].at[pl.ds(0, 128)]], rows_v.at[g],
                gsems[g],
            ).wait()

        iota16 = jnp.arange(16, dtype=jnp.int32)

        def transpose(g, tb):
            @pl.loop(0, d)
            def _(dd):
                si, r = dd // 8, dd % 8
                ddv = jnp.full((16,), 0, jnp.int32) + dd
                for k in range(8):
                    tiles_v.at[tb, si, r, pl.ds(k * 16, 16)][...] = (
                        plsc.load_gather(rows_v.at[g], [iota16 + k * 16, ddv])
                    )

        def transpose_fast(g, tb):
            for si in range(nd):
                for r in range(8):
                    ddv = jnp.full((16,), si * 8 + r, jnp.int32)
                    for k in range(8):
                        tiles_v.at[tb, si, r, pl.ds(k * 16, 16)][...] = (
                            plsc.load_gather(rows_v.at[g], [iota16 + k * 16, ddv])
                        )

        def write_out(u, tb):
            slot, j = u // lpw, u % lpw
            for si in range(nd):
                pltpu.async_copy(
                    tiles_v.at[tb, si], out_hbm.at[slot, si, lbase + j],
                    osems[tb],
                )

        def wait_out(tb):
            for si in range(nd):
                pltpu.make_async_copy(
                    tiles_v.at[tb, si], out_hbm.at[0, si, 0], osems[tb]
                ).wait()

        for g in range(_G):
            issue_gather(g, g)
        for g in range(_G):
            drain_gather(g)
            if g >= 2:
                wait_out(g % 2)
            transpose(g, g % 2)
            issue_gather(g + _G, g)
            write_out(g, g % 2)

        @pl.loop(_G, nunits - _G, step=_G)
        def _(u):
            for g in range(_G):
                drain_gather(g)
                wait_out(g % 2)
                transpose(g, g % 2)
                issue_gather(u + g + _G, g)
                write_out(u + g, g % 2)

        for g in range(_G):
            drain_gather(g)
            wait_out(g % 2)
            transpose(g, g % 2)
            write_out(nunits - _G + g, g % 2)
        wait_out(0)
        wait_out(1)

    kout = _gather(table_rm, idx_t)
    return kout.transpose(2, 4, 0, 1, 3).reshape(b, s, d)


# speedup vs baseline: 1.2501x; 1.2501x over previous
import functools

import jax
import jax.numpy as jnp
from jax import lax
from jax.experimental import pallas as pl
from jax.experimental.pallas import tpu as pltpu
from jax.experimental.pallas import tpu_sc as plsc

_NC = 2
_NS = 16
_NW = _NC * _NS
_G = 8


def kernel(indices, table):
    b, s = indices.shape
    d = table.shape[1]
    nd = d // 8
    nl = b // 128
    lpw = nl // _NW
    nunits = s * lpw
    idx_t = indices.T
    mesh = plsc.VectorSubcoreMesh(core_axis_name="c", subcore_axis_name="s")

    @functools.partial(
        pl.kernel,
        mesh=mesh,
        compiler_params=pltpu.CompilerParams(
            use_tc_tiling_on_sc=False, needs_layout_passes=False
        ),
        out_type=jax.ShapeDtypeStruct((s, nd, nl, 8, 128), table.dtype),
        scratch_types=[
            pltpu.VMEM((s, lpw * 128), jnp.int32),
            pltpu.VMEM((_G, 128, d), jnp.float32),
            pltpu.VMEM((2, nd, 8, 128), jnp.float32),
        ]
        + [pltpu.SemaphoreType.DMA] * (_G + 2),
    )
    def _gather(table_hbm, idx_hbm, out_hbm, idx_v, rows_v, tiles_v, *sems):
        gsems = sems[:_G]
        osems = sems[_G:]
        wid = lax.axis_index("c") * _NS + lax.axis_index("s")
        lbase = wid * lpw
        pltpu.sync_copy(idx_hbm.at[:, pl.ds(lbase * 128, lpw * 128)], idx_v)

        def issue_gather(u, g):
            slot, j = u // lpw, u % lpw
            pltpu.async_copy(
                table_hbm.at[idx_v.at[slot].at[pl.ds(j * 128, 128)]],
                rows_v.at[g],
                gsems[g],
            )

        def drain_gather(g):
            pltpu.make_async_copy(
                table_hbm.at[idx_v.at[0].at[pl.ds(0, 128)]], rows_v.at[g],
                gsems[g],
            ).wait()

        iota16 = jnp.arange(16, dtype=jnp.int32)

        def transpose(g, tb):
            @pl.loop(0, d)
            def _(dd):
                si, r = dd // 8, dd % 8
                ddv = jnp.full((16,), 0, jnp.int32) + dd
                for k in range(8):
                    tiles_v.at[tb, si, r, pl.ds(k * 16, 16)][...] = (
                        plsc.load_gather(rows_v.at[g], [iota16 + k * 16, ddv])
                    )

        def write_out(u, tb):
            slot, j = u // lpw, u % lpw
            for si in range(nd):
                pltpu.async_copy(
                    tiles_v.at[tb, si], out_hbm.at[slot, si, lbase + j],
                    osems[tb],
                )

        def wait_out(tb):
            for si in range(nd):
                pltpu.make_async_copy(
                    tiles_v.at[tb, si], out_hbm.at[0, si, 0], osems[tb]
                ).wait()

        for g in range(_G):
            issue_gather(g, g)
        for g in range(_G):
            drain_gather(g)
            if g >= 2:
                wait_out(g % 2)
            transpose(g, g % 2)
            issue_gather(g + _G, g)
            write_out(g, g % 2)

        @pl.loop(_G, nunits - _G, step=_G)
        def _(u):
            for g in range(_G):
                drain_gather(g)
                wait_out(g % 2)
                transpose(g, g % 2)
                issue_gather(u + g + _G, g)
                write_out(u + g, g % 2)

        for g in range(_G):
            drain_gather(g)
            wait_out(g % 2)
            transpose(g, g % 2)
            write_out(nunits - _G + g, g % 2)
        wait_out(0)
        wait_out(1)

    kout = _gather(table, idx_t)
    return kout.transpose(2, 4, 0, 1, 3).reshape(b, s, d)
